# baseline (speedup 1.0000x reference)
import functools

import jax
import jax.numpy as jnp
from jax import lax
from jax.experimental import pallas as pl
from jax.experimental.pallas import tpu as pltpu
from jax.experimental.pallas import tpu_sc as plsc

_NC, _NS = 2, 16
_NW = _NC * _NS
_SB = 4
_K = 48
_RC = 128


def _tc_body(scale_ref, x_ref, o_ref):
    i = pl.program_id(0)
    for j in range(_SB):
        o_ref[j] = x_ref[j] * scale_ref[i * _SB + j]


def _sc_body(x_hbm, sbc_hbm, out_hbm, buf, svec, C, H, W, Bsc):
    wid = lax.axis_index("s") * _NC + lax.axis_index("c")
    nrb = H // _RC
    cps = C * nrb
    upw = (Bsc * cps) // _NW
    lanes_per_row = W // 16
    for u in range(upw):
        g = u * _NW + wid
        b = g // cps
        rem = g % cps
        cc = rem // nrb
        rb = rem % nrb
        pltpu.sync_copy(sbc_hbm.at[b], svec)
        sv = svec[...]
        pltpu.sync_copy(x_hbm.at[_K + b, cc, pl.ds(rb * _RC, _RC), :], buf)

        def row_body(i, _, sv=sv):
            for j in range(lanes_per_row):
                sl = pl.ds(j * 16, 16)
                buf[i, sl] = buf[i, sl] * sv
            return 0

        lax.fori_loop(0, _RC, row_body, 0)
        pltpu.sync_copy(buf, out_hbm.at[b, cc, pl.ds(rb * _RC, _RC), :])


def kernel(input, probs, magnitudes):
    B, C, H, W = input.shape
    scale = jnp.where(probs, magnitudes, jnp.float32(1.0))
    Bsc = B - _K

    out_tc = pl.pallas_call(
        _tc_body,
        grid_spec=pltpu.PrefetchScalarGridSpec(
            num_scalar_prefetch=1,
            grid=(_K // _SB,),
            in_specs=[pl.BlockSpec((_SB, C, H, W), lambda i, s: (i, 0, 0, 0))],
            out_specs=pl.BlockSpec((_SB, C, H, W), lambda i, s: (i, 0, 0, 0)),
        ),
        out_shape=jax.ShapeDtypeStruct((_K, C, H, W), jnp.float32),
    )(scale, input)

    sbc = jnp.broadcast_to(scale[_K:, None], (Bsc, 16))
    sc_k = pl.kernel(
        functools.partial(_sc_body, C=C, H=H, W=W, Bsc=Bsc),
        out_type=jax.ShapeDtypeStruct((Bsc, C, H, W), jnp.float32),
        mesh=plsc.VectorSubcoreMesh(core_axis_name="c", subcore_axis_name="s"),
        scratch_types=[
            pltpu.VMEM((_RC, W), jnp.float32),
            pltpu.VMEM((16,), jnp.float32),
        ],
    )
    out_sc = sc_k(input, sbc)

    return jnp.concatenate([out_tc, out_sc], axis=0)


# speedup vs baseline: 2.1134x; 2.1134x over previous
import jax
import jax.numpy as jnp
from jax.experimental import pallas as pl
from jax.experimental.pallas import tpu as pltpu

_NBUF = 8
_LA = 4


def _plane(k, C, H, W):
    return (k // C, k % C)


def _body(scale_ref, x_hbm, o_hbm, bufs, in_sems, out_sems, *, C, H, W):
    k = pl.program_id(0)
    n = pl.num_programs(0)

    def start_in(i):
        b, c = _plane(i, C, H, W)
        s = i % _NBUF
        pltpu.make_async_copy(x_hbm.at[b, c], bufs.at[s], in_sems.at[s]).start()

    def wait_in(i):
        b, c = _plane(i, C, H, W)
        s = i % _NBUF
        pltpu.make_async_copy(x_hbm.at[b, c], bufs.at[s], in_sems.at[s]).wait()

    def start_out(i):
        b, c = _plane(i, C, H, W)
        s = i % _NBUF
        pltpu.make_async_copy(bufs.at[s], o_hbm.at[b, c], out_sems.at[s]).start()

    def wait_out(i):
        b, c = _plane(i, C, H, W)
        s = i % _NBUF
        pltpu.make_async_copy(bufs.at[s], o_hbm.at[b, c], out_sems.at[s]).wait()

    @pl.when(k == 0)
    def _prime():
        for j in range(_LA):
            start_in(j)

    @pl.when(jnp.logical_and(k >= _NBUF - _LA, k + _LA < n))
    def _retire():
        wait_out(k + _LA - _NBUF)

    @pl.when(k + _LA < n)
    def _prefetch():
        start_in(k + _LA)

    wait_in(k)
    s = k % _NBUF
    bufs[s] = bufs[s] * scale_ref[k // C]
    start_out(k)

    @pl.when(k == n - 1)
    def _drain():
        for j in range(_NBUF):
            wait_out(n - _NBUF + j)


def kernel(input, probs, magnitudes):
    B, C, H, W = input.shape
    scale = jnp.where(probs, magnitudes, jnp.float32(1.0))
    import functools
    body = functools.partial(_body, C=C, H=H, W=W)
    out = pl.pallas_call(
        body,
        grid_spec=pltpu.PrefetchScalarGridSpec(
            num_scalar_prefetch=1,
            grid=(B * C,),
            in_specs=[pl.BlockSpec(memory_space=pl.ANY)],
            out_specs=pl.BlockSpec(memory_space=pl.ANY),
            scratch_shapes=[
                pltpu.VMEM((_NBUF, H, W), jnp.float32),
                pltpu.SemaphoreType.DMA((_NBUF,)),
                pltpu.SemaphoreType.DMA((_NBUF,)),
            ],
        ),
        out_shape=jax.ShapeDtypeStruct((B, C, H, W), jnp.float32),
    )(scale, input)
    return out


# speedup vs baseline: 2.1162x; 1.0014x over previous
import functools

import jax
import jax.numpy as jnp
from jax.experimental import pallas as pl
from jax.experimental.pallas import tpu as pltpu

_NBUF = 4
_LA = 2


def _body(probs_ref, mags_ref, x_hbm, o_hbm, bufs, in_sems, out_sems):
    k = pl.program_id(0)
    n = pl.num_programs(0)

    def start_in(i):
        s = i % _NBUF
        pltpu.make_async_copy(x_hbm.at[i], bufs.at[s], in_sems.at[s]).start()

    def wait_in(i):
        s = i % _NBUF
        pltpu.make_async_copy(x_hbm.at[i], bufs.at[s], in_sems.at[s]).wait()

    def start_out(i):
        s = i % _NBUF
        pltpu.make_async_copy(bufs.at[s], o_hbm.at[i], out_sems.at[s]).start()

    def wait_out(i):
        s = i % _NBUF
        pltpu.make_async_copy(bufs.at[s], o_hbm.at[i], out_sems.at[s]).wait()

    @pl.when(k == 0)
    def _prime():
        for j in range(_LA):
            start_in(j)

    @pl.when(jnp.logical_and(k >= _NBUF - _LA, k + _LA < n))
    def _retire():
        wait_out(k + _LA - _NBUF)

    @pl.when(k + _LA < n)
    def _prefetch():
        start_in(k + _LA)

    wait_in(k)
    s = k % _NBUF
    scale = jnp.where(probs_ref[k] != 0, mags_ref[k], jnp.float32(1.0))
    bufs[s] = bufs[s] * scale
    start_out(k)

    @pl.when(k == n - 1)
    def _drain():
        for j in range(_NBUF):
            wait_out(n - _NBUF + j)


def kernel(input, probs, magnitudes):
    B, C, H, W = input.shape
    out = pl.pallas_call(
        _body,
        grid_spec=pltpu.PrefetchScalarGridSpec(
            num_scalar_prefetch=2,
            grid=(B,),
            in_specs=[pl.BlockSpec(memory_space=pl.ANY)],
            out_specs=pl.BlockSpec(memory_space=pl.ANY),
            scratch_shapes=[
                pltpu.VMEM((_NBUF, C, H, W), jnp.float32),
                pltpu.SemaphoreType.DMA((_NBUF,)),
                pltpu.SemaphoreType.DMA((_NBUF,)),
            ],
        ),
        out_shape=jax.ShapeDtypeStruct((B, C, H, W), jnp.float32),
    )(probs, magnitudes, input)
    return out


# speedup vs baseline: 2.1350x; 1.0089x over previous
import jax
import jax.numpy as jnp
from jax.experimental import pallas as pl
from jax.experimental.pallas import tpu as pltpu

_SB = 4


def _scale_body(probs_ref, mags_ref, x_ref, o_ref):
    i = pl.program_id(0)
    for j in range(_SB):
        b = i * _SB + j
        scale = jnp.where(probs_ref[b] != 0, mags_ref[b], jnp.float32(1.0))
        o_ref[j] = x_ref[j] * scale


def kernel(input, probs, magnitudes):
    B, C, H, W = input.shape
    out = pl.pallas_call(
        _scale_body,
        grid_spec=pltpu.PrefetchScalarGridSpec(
            num_scalar_prefetch=2,
            grid=(B // _SB,),
            in_specs=[pl.BlockSpec((_SB, C, H, W), lambda i, p, m: (i, 0, 0, 0))],
            out_specs=pl.BlockSpec((_SB, C, H, W), lambda i, p, m: (i, 0, 0, 0)),
        ),
        out_shape=jax.ShapeDtypeStruct((B, C, H, W), jnp.float32),
    )(probs, magnitudes, input)
    return out
